# baseline (speedup 1.0000x reference)
import jax
import jax.numpy as jnp
from jax import lax
from jax.experimental import pallas as pl
from jax.experimental.pallas import tpu as pltpu
from jax.experimental.pallas import tpu_sc as plsc

_B, _H, _S, _D = 32, 8, 2048, 64
_CH = 256
_NCH = _S // _CH


def _sc_body(k_hbm, v_hbm, nk_hbm, nv_hbm, par_hbm, out_hbm,
             buf0, buf1, par_v, new_v, sg0, sg1, ss0, ss1):
    w = lax.axis_index("s") * 2 + lax.axis_index("c")
    pltpu.sync_copy(par_hbm.at[w], par_v)
    par = par_v[...]
    src = par[0]
    p = par[1]
    pltpu.sync_copy(nk_hbm.at[src, :, 0, :], new_v.at[pl.ds(0, _H)])
    pltpu.sync_copy(nv_hbm.at[src, :, 0, :], new_v.at[pl.ds(_H, _H)])

    bufs = (buf0, buf1)
    gsem = (sg0, sg1)
    ssem = (ss0, ss1)
    pend = [None, None]

    def step(c, src_ref, kv, h, s0):
        slot = c % 2
        if pend[slot] is not None:
            pend[slot].wait()
        g = pltpu.make_async_copy(src_ref.at[w, h, pl.ds(s0, _CH), :],
                                  bufs[slot], gsem[slot])
        g.start()
        g.wait()
        wr = pltpu.make_async_copy(bufs[slot],
                                   out_hbm.at[kv, w, h, pl.ds(s0, _CH), :],
                                   ssem[slot])
        wr.start()
        pend[slot] = wr

    c = 0
    for kv, src_ref in ((0, k_hbm), (1, v_hbm)):
        for h in range(_H):
            for j in range(_NCH):
                step(c, src_ref, kv, h, j * _CH)
                c += 1
    for wr in pend:
        if wr is not None:
            wr.wait()
    for kv in range(2):
        for h in range(_H):
            pltpu.sync_copy(new_v.at[pl.ds(kv * _H + h, 1)],
                            out_hbm.at[kv, w, h, pl.ds(p, 1), :])


def kernel(k_cache, v_cache, new_k, new_v, seq_ids, position_ids):
    b, h, s, d = k_cache.shape
    inv = jnp.argsort(seq_ids).astype(jnp.int32)
    pos = position_ids[inv, 0].astype(jnp.int32)
    params = jnp.zeros((b, 16), jnp.int32)
    params = params.at[:, 0].set(inv).at[:, 1].set(pos)

    mesh = plsc.VectorSubcoreMesh(core_axis_name="c", subcore_axis_name="s")
    run = pl.kernel(
        _sc_body,
        mesh=mesh,
        out_type=jax.ShapeDtypeStruct((2, b, h, s, d), k_cache.dtype),
        scratch_types=[
            pltpu.VMEM((_CH, _D), jnp.float32),
            pltpu.VMEM((_CH, _D), jnp.float32),
            pltpu.VMEM((16,), jnp.int32),
            pltpu.VMEM((2 * _H, _D), jnp.float32),
            pltpu.SemaphoreType.DMA,
            pltpu.SemaphoreType.DMA,
            pltpu.SemaphoreType.DMA,
            pltpu.SemaphoreType.DMA,
        ],
    )
    return run(k_cache, v_cache, new_k, new_v, params)


# speedup vs baseline: 1.1703x; 1.1703x over previous
import jax
import jax.numpy as jnp
from jax import lax
from jax.experimental import pallas as pl
from jax.experimental.pallas import tpu as pltpu
from jax.experimental.pallas import tpu_sc as plsc

_B, _H, _S, _D = 32, 8, 2048, 64
_CH = 256
_NCH = _S // _CH


def _sc_body(k_hbm, v_hbm, nk_hbm, nv_hbm, par_hbm, out_hbm,
             buf0, buf1, par_v, new_v, sg0, sg1, ss0, ss1):
    w = lax.axis_index("s") * 2 + lax.axis_index("c")
    pltpu.sync_copy(par_hbm.at[w], par_v)
    par = par_v[...]
    src = par[0]
    p = par[1]
    pltpu.sync_copy(nk_hbm.at[src, :, 0, :], new_v.at[pl.ds(0, _H)])
    pltpu.sync_copy(nv_hbm.at[src, :, 0, :], new_v.at[pl.ds(_H, _H)])

    bufs = (buf0, buf1)
    gsem = (sg0, sg1)
    ssem = (ss0, ss1)
    pend = [None, None]

    def step(c, src_ref, kv, h, s0):
        slot = c % 2
        if pend[slot] is not None:
            pend[slot].wait()
        g = pltpu.make_async_copy(
            src_ref.at[pl.ds((w * _H + h) * _S + s0, _CH)], bufs[slot],
            gsem[slot])
        g.start()
        g.wait()
        wr = pltpu.make_async_copy(bufs[slot],
                                   out_hbm.at[kv, w, h, pl.ds(s0, _CH), :],
                                   ssem[slot])
        wr.start()
        pend[slot] = wr

    c = 0
    for kv, src_ref in ((0, k_hbm), (1, v_hbm)):
        for h in range(_H):
            for j in range(_NCH):
                step(c, src_ref, kv, h, j * _CH)
                c += 1
    for wr in pend:
        if wr is not None:
            wr.wait()
    for kv in range(2):
        for h in range(_H):
            pltpu.sync_copy(new_v.at[pl.ds(kv * _H + h, 1)],
                            out_hbm.at[kv, w, h, pl.ds(p, 1), :])


def kernel(k_cache, v_cache, new_k, new_v, seq_ids, position_ids):
    b, h, s, d = k_cache.shape
    inv = jnp.argsort(seq_ids).astype(jnp.int32)
    pos = position_ids[inv, 0].astype(jnp.int32)
    params = jnp.zeros((b, 16), jnp.int32)
    params = params.at[:, 0].set(inv).at[:, 1].set(pos)

    mesh = plsc.VectorSubcoreMesh(core_axis_name="c", subcore_axis_name="s")
    run = pl.kernel(
        _sc_body,
        mesh=mesh,
        out_type=jax.ShapeDtypeStruct((2, b, h, s, d), k_cache.dtype),
        scratch_types=[
            pltpu.VMEM((_CH, _D), jnp.float32),
            pltpu.VMEM((_CH, _D), jnp.float32),
            pltpu.VMEM((16,), jnp.int32),
            pltpu.VMEM((2 * _H, _D), jnp.float32),
            pltpu.SemaphoreType.DMA,
            pltpu.SemaphoreType.DMA,
            pltpu.SemaphoreType.DMA,
            pltpu.SemaphoreType.DMA,
        ],
    )
    k64 = k_cache.reshape(b * h * s, d)
    v64 = v_cache.reshape(b * h * s, d)
    return run(k64, v64, new_k, new_v, params)


# speedup vs baseline: 1.4299x; 1.2219x over previous
import jax
import jax.numpy as jnp
from jax import lax
from jax.experimental import pallas as pl
from jax.experimental.pallas import tpu as pltpu
from jax.experimental.pallas import tpu_sc as plsc

_B, _H, _S, _D = 32, 8, 2048, 64
_CH = 256
_NCH = _S // _CH
_NB = 3


def _sc_body(k64, v64, nk_hbm, nv_hbm, par_hbm, out64,
             buf0, buf1, buf2, par_v, new_v,
             sg0, sg1, sg2, ss0, ss1, ss2):
    w = lax.axis_index("s") * 2 + lax.axis_index("c")
    pltpu.sync_copy(par_hbm.at[w], par_v)
    par = par_v[...]
    src = par[0]
    p = par[1]
    pltpu.sync_copy(nk_hbm.at[src, :, 0, :], new_v.at[pl.ds(0, _H)])
    pltpu.sync_copy(nv_hbm.at[src, :, 0, :], new_v.at[pl.ds(_H, _H)])

    bufs = (buf0, buf1, buf2)
    gsem = (sg0, sg1, sg2)
    ssem = (ss0, ss1, ss2)
    g_pend = [None] * _NB
    s_pend = [None] * _NB

    def flush(slot):
        if g_pend[slot] is not None:
            g, dst = g_pend[slot]
            g.wait()
            s = pltpu.make_async_copy(bufs[slot], out64.at[pl.ds(dst, _CH)],
                                      ssem[slot])
            s.start()
            s_pend[slot] = s
            g_pend[slot] = None

    def step(c, src_ref, src_line, dst_line):
        slot = c % _NB
        if s_pend[slot] is not None:
            s_pend[slot].wait()
            s_pend[slot] = None
        g = pltpu.make_async_copy(src_ref.at[pl.ds(src_line, _CH)],
                                  bufs[slot], gsem[slot])
        g.start()
        g_pend[slot] = (g, dst_line)
        flush((c - 1) % _NB)

    vout = _B * _H * _S
    c = 0
    for kv, src_ref in ((0, k64), (1, v64)):
        for h in range(_H):
            for j in range(_NCH):
                line = (w * _H + h) * _S + j * _CH
                step(c, src_ref, line, kv * vout + line)
                c += 1
    for slot in range(_NB):
        flush(slot)
    for s in s_pend:
        if s is not None:
            s.wait()
    for kv in range(2):
        for h in range(_H):
            line = kv * vout + (w * _H + h) * _S + p
            pltpu.sync_copy(new_v.at[pl.ds(kv * _H + h, 1)],
                            out64.at[pl.ds(line, 1)])


def kernel(k_cache, v_cache, new_k, new_v, seq_ids, position_ids):
    b, h, s, d = k_cache.shape
    inv = jnp.argsort(seq_ids).astype(jnp.int32)
    pos = position_ids[inv, 0].astype(jnp.int32)
    params = jnp.zeros((b, 16), jnp.int32)
    params = params.at[:, 0].set(inv).at[:, 1].set(pos)
    k64 = k_cache.reshape(b * h * s, d)
    v64 = v_cache.reshape(b * h * s, d)

    mesh = plsc.VectorSubcoreMesh(core_axis_name="c", subcore_axis_name="s")
    run = pl.kernel(
        _sc_body,
        mesh=mesh,
        out_type=jax.ShapeDtypeStruct((2 * b * h * s, d), k_cache.dtype),
        scratch_types=[
            pltpu.VMEM((_CH, _D), jnp.float32),
            pltpu.VMEM((_CH, _D), jnp.float32),
            pltpu.VMEM((_CH, _D), jnp.float32),
            pltpu.VMEM((16,), jnp.int32),
            pltpu.VMEM((2 * _H, _D), jnp.float32),
            pltpu.SemaphoreType.DMA,
            pltpu.SemaphoreType.DMA,
            pltpu.SemaphoreType.DMA,
            pltpu.SemaphoreType.DMA,
            pltpu.SemaphoreType.DMA,
            pltpu.SemaphoreType.DMA,
        ],
    )
    out = run(k64, v64, new_k, new_v, params)
    return out.reshape(2, b, h, s, d)
